# Initial kernel scaffold; baseline (speedup 1.0000x reference)
import jax
import jax.numpy as jnp
from jax import lax
from jax.experimental import pallas as pl
from jax.experimental.pallas import tpu as pltpu
from jax.experimental.pallas import tpu_sc as plsc

L = 16
NC = 2
NS = 16
NW = NC * NS
CHUNK = 128
N_COMP = 5
C_DIM = 64
COMP_PAD = 16


def _tce_body(x_hbm, comp_hbm, e0, e1, e2, e3, e4,
              o0, o1, o2, o3, o4,
              x_v, comps_v, idx_v, rows_v, sem):
    embs = (e0, e1, e2, e3, e4)
    outs = (o0, o1, o2, o3, o4)
    batch = x_hbm.shape[0]
    per_w = batch // NW
    nchunks = per_w // CHUNK

    wid = lax.axis_index("s") * NC + lax.axis_index("c")

    for c in range(nchunks):
        gbase = wid * per_w + c * CHUNK
        pltpu.sync_copy(x_hbm.at[pl.ds(gbase, CHUNK)], x_v)
        pltpu.async_copy(comp_hbm.at[x_v], comps_v, sem).wait()
        for j in range(CHUNK // L):
            r = jnp.arange(L, dtype=jnp.int32) + (j * L)
            for i in range(N_COMP):
                ci = jnp.full((L,), i, dtype=jnp.int32)
                idx_v[i][pl.ds(j * L, L)] = plsc.load_gather(comps_v, [r, ci])
        for i in range(N_COMP):
            pltpu.async_copy(embs[i].at[idx_v[i]], rows_v, sem).wait()
            pltpu.sync_copy(rows_v, outs[i].at[pl.ds(gbase, CHUNK)])


def kernel(x, comp_table, emb0, emb1, emb2, emb3, emb4):
    batch = x.shape[0]
    t_vocab = comp_table.shape[0]
    comp_pad = jnp.zeros((t_vocab, COMP_PAD), jnp.int32).at[:, :N_COMP].set(comp_table)
    embs = tuple(e.at[0].set(0.0) for e in (emb0, emb1, emb2, emb3, emb4))

    mesh = plsc.VectorSubcoreMesh(core_axis_name="c", subcore_axis_name="s")
    out_type = tuple(
        jax.ShapeDtypeStruct((batch, C_DIM), jnp.float32) for _ in range(N_COMP)
    )
    scratch = [
        pltpu.VMEM((CHUNK,), jnp.int32),
        pltpu.VMEM((CHUNK, COMP_PAD), jnp.int32),
        [pltpu.VMEM((CHUNK,), jnp.int32) for _ in range(N_COMP)],
        pltpu.VMEM((CHUNK, C_DIM), jnp.float32),
        pltpu.SemaphoreType.DMA,
    ]
    f = pl.kernel(_tce_body, mesh=mesh, out_type=out_type, scratch_types=scratch)
    return f(x, comp_pad, *embs)


# speedup vs baseline: 1.6358x; 1.6358x over previous
import jax
import jax.numpy as jnp
from jax import lax
from jax.experimental import pallas as pl
from jax.experimental.pallas import tpu as pltpu
from jax.experimental.pallas import tpu_sc as plsc

L = 16
NC = 2
NS = 16
NW = NC * NS
CHUNK = 128
N_COMP = 5
C_DIM = 64


def _tce_body(x_hbm, comp_hbm, e0, e1, e2, e3, e4,
              o0, o1, o2, o3, o4,
              x_v, cidx_v, cvals_v, rows_v, sem):
    embs = (e0, e1, e2, e3, e4)
    outs = (o0, o1, o2, o3, o4)
    batch = x_hbm.shape[0]
    t_vocab = comp_hbm.shape[0] // N_COMP
    per_w = batch // NW
    nchunks = per_w // CHUNK

    wid = lax.axis_index("s") * NC + lax.axis_index("c")

    for c in range(nchunks):
        gbase = wid * per_w + c * CHUNK
        pltpu.sync_copy(x_hbm.at[pl.ds(gbase, CHUNK)], x_v)
        for i in range(N_COMP):
            off = jnp.int32(i * t_vocab)
            for j in range(CHUNK // L):
                sl = pl.ds(j * L, L)
                cidx_v[sl] = x_v[sl] + off
            pltpu.async_copy(comp_hbm.at[cidx_v], cvals_v, sem).wait()
            pltpu.async_copy(embs[i].at[cvals_v], rows_v, sem).wait()
            pltpu.sync_copy(rows_v, outs[i].at[pl.ds(gbase, CHUNK)])


def kernel(x, comp_table, emb0, emb1, emb2, emb3, emb4):
    batch = x.shape[0]
    comp_cm = comp_table.T.reshape(-1)
    embs = tuple(e.at[0].set(0.0) for e in (emb0, emb1, emb2, emb3, emb4))

    mesh = plsc.VectorSubcoreMesh(core_axis_name="c", subcore_axis_name="s")
    out_type = tuple(
        jax.ShapeDtypeStruct((batch, C_DIM), jnp.float32) for _ in range(N_COMP)
    )
    scratch = [
        pltpu.VMEM((CHUNK,), jnp.int32),
        pltpu.VMEM((CHUNK,), jnp.int32),
        pltpu.VMEM((CHUNK,), jnp.int32),
        pltpu.VMEM((CHUNK, C_DIM), jnp.float32),
        pltpu.SemaphoreType.DMA,
    ]
    f = pl.kernel(
        _tce_body, mesh=mesh, out_type=out_type, scratch_types=scratch,
        compiler_params=pltpu.CompilerParams(use_tc_tiling_on_sc=False),
    )
    return f(x, comp_cm, *embs)
